# baseline (speedup 1.0000x reference)
import functools

import jax
import jax.numpy as jnp
from jax import lax
from jax.experimental import pallas as pl
from jax.experimental.pallas import tpu as pltpu
from jax.experimental.pallas import tpu_sc as plsc

NC, NS, LANES = 2, 16, 16
NW = NC * NS

N_NODES = 10000
D = 128
N_EDGES = 320000
NUM_ITERS = 10

PAD_ROWS = 64
NH = N_NODES + PAD_ROWS
NA = 10112
ROWS_PER_TILE = NA // NS

CHUNK = 184
NCHUNKS = 55
EP = NW * NCHUNKS * CHUNK
NSLOT = 2
NIB = 4
PERIOD = 4

_mesh = plsc.VectorSubcoreMesh(core_axis_name="c", subcore_axis_name="s")


@functools.partial(
    pl.kernel,
    out_type=jax.ShapeDtypeStruct((NC, NA, D), jnp.float32),
    mesh=_mesh,
    scratch_types=[
        pltpu.VMEM_SHARED((NA, D), jnp.float32),
        pltpu.VMEM((NSLOT * CHUNK, D), jnp.float32),
        [pltpu.VMEM((CHUNK,), jnp.int32)] * NIB,
        [pltpu.VMEM((CHUNK,), jnp.int32)] * NIB,
        [pltpu.SemaphoreType.DMA] * NSLOT,
        [pltpu.SemaphoreType.DMA] * NSLOT,
        [pltpu.SemaphoreType.DMA] * NIB,
    ],
)
def _sc_round(h_hbm, src_hbm, dst_hbm, out_hbm, agg, rows, sidx, didx,
              sem_g, sem_s, sem_i):
    c = lax.axis_index("c")
    s = lax.axis_index("s")
    w = s * NC + c
    base = s * ROWS_PER_TILE

    slots = [rows.at[pl.ds(k * CHUNK, CHUNK)] for k in range(NSLOT)]

    zv = jnp.zeros((LANES,), jnp.float32)

    def zero_body(i, carry):
        for k in range(D // LANES):
            rows[i, pl.ds(k * LANES, LANES)] = zv
        return carry

    lax.fori_loop(0, 64, zero_body, 0)
    for k in range(9):
        pltpu.sync_copy(rows.at[pl.ds(0, 64)], agg.at[pl.ds(base + k * 64, 64)])
    pltpu.sync_copy(rows.at[pl.ds(0, 56)], agg.at[pl.ds(base + 576, 56)])
    plsc.subcore_barrier()

    def off(i):
        return (w * NCHUNKS + i) * CHUNK

    def load_idx(i, q):
        pltpu.async_copy(src_hbm.at[pl.ds(off(i), CHUNK)], sidx[q], sem_i[q])
        pltpu.async_copy(dst_hbm.at[pl.ds(off(i), CHUNK)], didx[q], sem_i[q])

    def gather(q, p):
        pltpu.async_copy(h_hbm.at[sidx[q]], slots[p], sem_g[p])

    def scatter(q, p):
        pltpu.async_copy(slots[p], agg.at[didx[q]], sem_s[p], add=True)

    def drain_g(p):
        pltpu.make_async_copy(
            h_hbm.at[pl.ds(0, CHUNK)], slots[p], sem_g[p]
        ).wait()

    def drain_s(p):
        pltpu.make_async_copy(
            h_hbm.at[pl.ds(0, CHUNK)], slots[p], sem_s[p]
        ).wait()

    def drain_i(q):
        for _ in range(2):
            pltpu.make_async_copy(
                src_hbm.at[pl.ds(0, CHUNK)], sidx[q], sem_i[q]
            ).wait()

    def stage(i, jm, first_s=False, idx_next=True, gather_next=True):
        if not first_s:
            drain_s((jm - 1) % NSLOT)
        if gather_next:
            drain_i((jm + NSLOT - 1) % NIB)
            gather((jm + NSLOT - 1) % NIB, (jm + NSLOT - 1) % NSLOT)
        drain_g(jm % NSLOT)
        if idx_next:
            load_idx(i + NSLOT, (jm + NSLOT) % NIB)
        scatter(jm % NIB, jm % NSLOT)

    for i in range(NSLOT - 1):
        pltpu.sync_copy(src_hbm.at[pl.ds(off(i), CHUNK)], sidx[i])
        pltpu.sync_copy(dst_hbm.at[pl.ds(off(i), CHUNK)], didx[i])
        gather(i, i)
    load_idx(NSLOT - 1, NSLOT - 1)

    for i in range(3):
        stage(i, i % PERIOD, first_s=(i == 0))

    def loop_body(k, carry):
        for jj in range(PERIOD):
            stage(3 + PERIOD * k + jj, (3 + jj) % PERIOD)
        return carry

    lax.fori_loop(0, 12, loop_body, 0)

    for i in range(51, NCHUNKS):
        stage(
            i,
            i % PERIOD,
            idx_next=(i + NSLOT < NCHUNKS),
            gather_next=(i + NSLOT - 1 < NCHUNKS),
        )
    drain_s((NCHUNKS - 1) % NSLOT)
    plsc.subcore_barrier()

    pltpu.sync_copy(
        agg.at[pl.ds(base, ROWS_PER_TILE)],
        out_hbm.at[c, pl.ds(base, ROWS_PER_TILE)],
    )


def _combine_body(p0_ref, p1_ref, d0_ref, d1_ref, o_ref):
    scale = 1.0 / jnp.maximum(d0_ref[...] + d1_ref[...], 1.0)
    o_ref[pl.ds(0, N_NODES), :] = (
        p0_ref[pl.ds(0, N_NODES), :] + p1_ref[pl.ds(0, N_NODES), :]
    ) * scale
    o_ref[pl.ds(N_NODES, PAD_ROWS), :] = jnp.zeros(
        (PAD_ROWS, D), jnp.float32
    )


_combine = pl.pallas_call(
    _combine_body,
    out_shape=jax.ShapeDtypeStruct((NH, D), jnp.float32),
)


def kernel(x, edge_index):
    src = edge_index[0].astype(jnp.int32)
    dst = edge_index[1].astype(jnp.int32)
    npad = EP - N_EDGES
    pad_src = N_NODES + (lax.iota(jnp.int32, npad) % PAD_ROWS)
    pad_dst = lax.iota(jnp.int32, npad) % N_NODES
    srcp = jnp.concatenate([src, pad_src])
    dstp = jnp.concatenate([dst, pad_dst])

    h = jnp.concatenate([x, jnp.zeros((PAD_ROWS, D), jnp.float32)], axis=0)

    ones_h = jnp.concatenate(
        [
            jnp.ones((N_NODES, D), jnp.float32),
            jnp.zeros((PAD_ROWS, D), jnp.float32),
        ],
        axis=0,
    )
    dp = _sc_round(ones_h, srcp, dstp)
    d0 = dp[0, :N_NODES, 0:1]
    d1 = dp[1, :N_NODES, 0:1]

    for _ in range(NUM_ITERS):
        p = _sc_round(h, srcp, dstp)
        h = _combine(p[0], p[1], d0, d1)

    return h[:N_NODES]


# speedup vs baseline: 1.0981x; 1.0981x over previous
import functools

import jax
import jax.numpy as jnp
from jax import lax
from jax.experimental import pallas as pl
from jax.experimental.pallas import tpu as pltpu
from jax.experimental.pallas import tpu_sc as plsc

NC, NS, LANES = 2, 16, 16
NW = NC * NS

N_NODES = 10000
D = 128
N_EDGES = 320000
NUM_ITERS = 10

PAD_ROWS = 64
NH = N_NODES + PAD_ROWS
NA = 10112
ROWS_PER_TILE = NA // NS

CHUNK = 120
NCHUNKS = 84
EP = NW * NCHUNKS * CHUNK
NSLOT = 3
NIB = 6
PERIOD = 6
IDXLA = 4

_mesh = plsc.VectorSubcoreMesh(core_axis_name="c", subcore_axis_name="s")


@functools.partial(
    pl.kernel,
    out_type=jax.ShapeDtypeStruct((NC, NA, D), jnp.float32),
    mesh=_mesh,
    scratch_types=[
        pltpu.VMEM_SHARED((NA, D), jnp.float32),
        pltpu.VMEM((NSLOT * CHUNK, D), jnp.float32),
        [pltpu.VMEM((CHUNK,), jnp.int32)] * NIB,
        [pltpu.VMEM((CHUNK,), jnp.int32)] * NIB,
        [pltpu.SemaphoreType.DMA] * NSLOT,
        [pltpu.SemaphoreType.DMA] * NSLOT,
        [pltpu.SemaphoreType.DMA] * NIB,
    ],
)
def _sc_round(h_hbm, src_hbm, dst_hbm, out_hbm, agg, rows, sidx, didx,
              sem_g, sem_s, sem_i):
    c = lax.axis_index("c")
    s = lax.axis_index("s")
    w = s * NC + c
    base = s * ROWS_PER_TILE

    slots = [rows.at[pl.ds(k * CHUNK, CHUNK)] for k in range(NSLOT)]

    zv = jnp.zeros((LANES,), jnp.float32)

    def zero_body(i, carry):
        for k in range(D // LANES):
            rows[i, pl.ds(k * LANES, LANES)] = zv
        return carry

    lax.fori_loop(0, 64, zero_body, 0)
    for k in range(9):
        pltpu.sync_copy(rows.at[pl.ds(0, 64)], agg.at[pl.ds(base + k * 64, 64)])
    pltpu.sync_copy(rows.at[pl.ds(0, 56)], agg.at[pl.ds(base + 576, 56)])
    plsc.subcore_barrier()

    def off(i):
        return (w * NCHUNKS + i) * CHUNK

    def load_idx(i, q):
        pltpu.async_copy(src_hbm.at[pl.ds(off(i), CHUNK)], sidx[q], sem_i[q])
        pltpu.async_copy(dst_hbm.at[pl.ds(off(i), CHUNK)], didx[q], sem_i[q])

    def gather(q, p):
        pltpu.async_copy(h_hbm.at[sidx[q]], slots[p], sem_g[p])

    def scatter(q, p):
        pltpu.async_copy(slots[p], agg.at[didx[q]], sem_s[p], add=True)

    def drain_g(p):
        pltpu.make_async_copy(
            h_hbm.at[pl.ds(0, CHUNK)], slots[p], sem_g[p]
        ).wait()

    def drain_s(p):
        pltpu.make_async_copy(
            h_hbm.at[pl.ds(0, CHUNK)], slots[p], sem_s[p]
        ).wait()

    def drain_i(q):
        for _ in range(2):
            pltpu.make_async_copy(
                src_hbm.at[pl.ds(0, CHUNK)], sidx[q], sem_i[q]
            ).wait()

    def stage(i, jm, first_s=False, idx_next=True, gather_next=True):
        if not first_s:
            drain_s((jm - 1) % NSLOT)
        if gather_next:
            drain_i((jm + NSLOT - 1) % NIB)
            gather((jm + NSLOT - 1) % NIB, (jm + NSLOT - 1) % NSLOT)
        drain_g(jm % NSLOT)
        if idx_next:
            load_idx(i + IDXLA, (jm + IDXLA) % NIB)
        scatter(jm % NIB, jm % NSLOT)

    for i in range(NSLOT - 1):
        pltpu.sync_copy(src_hbm.at[pl.ds(off(i), CHUNK)], sidx[i])
        pltpu.sync_copy(dst_hbm.at[pl.ds(off(i), CHUNK)], didx[i])
        gather(i, i)
    load_idx(NSLOT - 1, NSLOT - 1)
    load_idx(NSLOT, NSLOT)

    for i in range(2):
        stage(i, i % PERIOD, first_s=(i == 0))

    def loop_body(k, carry):
        for jj in range(PERIOD):
            stage(2 + PERIOD * k + jj, (2 + jj) % PERIOD)
        return carry

    lax.fori_loop(0, 12, loop_body, 0)

    for i in range(74, NCHUNKS):
        stage(
            i,
            i % PERIOD,
            idx_next=(i + IDXLA < NCHUNKS),
            gather_next=(i + NSLOT - 1 < NCHUNKS),
        )
    drain_s((NCHUNKS - 1) % NSLOT)
    plsc.subcore_barrier()

    pltpu.sync_copy(
        agg.at[pl.ds(base, ROWS_PER_TILE)],
        out_hbm.at[c, pl.ds(base, ROWS_PER_TILE)],
    )


def _combine_body(p0_ref, p1_ref, d0_ref, d1_ref, o_ref):
    scale = 1.0 / jnp.maximum(d0_ref[...] + d1_ref[...], 1.0)
    o_ref[pl.ds(0, N_NODES), :] = (
        p0_ref[pl.ds(0, N_NODES), :] + p1_ref[pl.ds(0, N_NODES), :]
    ) * scale
    o_ref[pl.ds(N_NODES, PAD_ROWS), :] = jnp.zeros(
        (PAD_ROWS, D), jnp.float32
    )


_combine = pl.pallas_call(
    _combine_body,
    out_shape=jax.ShapeDtypeStruct((NH, D), jnp.float32),
)


def kernel(x, edge_index):
    src = edge_index[0].astype(jnp.int32)
    dst = edge_index[1].astype(jnp.int32)
    npad = EP - N_EDGES
    pad_src = N_NODES + (lax.iota(jnp.int32, npad) % PAD_ROWS)
    pad_dst = lax.iota(jnp.int32, npad) % N_NODES
    srcp = jnp.concatenate([src, pad_src])
    dstp = jnp.concatenate([dst, pad_dst])

    h = jnp.concatenate([x, jnp.zeros((PAD_ROWS, D), jnp.float32)], axis=0)

    ones_h = jnp.concatenate(
        [
            jnp.ones((N_NODES, D), jnp.float32),
            jnp.zeros((PAD_ROWS, D), jnp.float32),
        ],
        axis=0,
    )
    dp = _sc_round(ones_h, srcp, dstp)
    d0 = dp[0, :N_NODES, 0:1]
    d1 = dp[1, :N_NODES, 0:1]

    for _ in range(NUM_ITERS):
        p = _sc_round(h, srcp, dstp)
        h = _combine(p[0], p[1], d0, d1)

    return h[:N_NODES]


# speedup vs baseline: 1.1038x; 1.0052x over previous
import functools

import jax
import jax.numpy as jnp
from jax import lax
from jax.experimental import pallas as pl
from jax.experimental.pallas import tpu as pltpu
from jax.experimental.pallas import tpu_sc as plsc

NC, NS, LANES = 2, 16, 16
NW = NC * NS

N_NODES = 10000
D = 128
N_EDGES = 320000
NUM_ITERS = 10

NA = 10112
PAD_ROWS = NA - N_NODES
ROWS_PER_TILE = NA // NS

CHUNK = 120
NCHUNKS = 84
EP = NW * NCHUNKS * CHUNK
NSLOT = 3
NIB = 6
PERIOD = 6
IDXLA = 4

_mesh = plsc.VectorSubcoreMesh(core_axis_name="c", subcore_axis_name="s")


@functools.partial(
    pl.kernel,
    out_type=jax.ShapeDtypeStruct((NC, NA, D), jnp.float32),
    mesh=_mesh,
    scratch_types=[
        pltpu.VMEM_SHARED((NA, D), jnp.float32),
        pltpu.VMEM((NSLOT * CHUNK, D), jnp.float32),
        [pltpu.VMEM((CHUNK,), jnp.int32)] * NIB,
        [pltpu.VMEM((CHUNK,), jnp.int32)] * NIB,
        [pltpu.SemaphoreType.DMA] * NSLOT,
        [pltpu.SemaphoreType.DMA] * NSLOT,
        [pltpu.SemaphoreType.DMA] * NIB,
    ],
)
def _sc_round(h_hbm, src_hbm, dst_hbm, out_hbm, agg, rows, sidx, didx,
              sem_g, sem_s, sem_i):
    c = lax.axis_index("c")
    s = lax.axis_index("s")
    w = s * NC + c
    base = s * ROWS_PER_TILE

    slots = [rows.at[pl.ds(k * CHUNK, CHUNK)] for k in range(NSLOT)]

    def off(i):
        return (w * NCHUNKS + i) * CHUNK

    def load_idx(i, q):
        pltpu.async_copy(src_hbm.at[pl.ds(off(i), CHUNK)], sidx[q], sem_i[q])
        pltpu.async_copy(dst_hbm.at[pl.ds(off(i), CHUNK)], didx[q], sem_i[q])

    def gather(q, p):
        pltpu.async_copy(h_hbm.at[sidx[q]], slots[p], sem_g[p])

    def scatter(q, p):
        pltpu.async_copy(slots[p], agg.at[didx[q]], sem_s[p], add=True)

    def drain_g(p):
        pltpu.make_async_copy(
            h_hbm.at[pl.ds(0, CHUNK)], slots[p], sem_g[p]
        ).wait()

    def drain_s(p):
        pltpu.make_async_copy(
            h_hbm.at[pl.ds(0, CHUNK)], slots[p], sem_s[p]
        ).wait()

    def drain_i(q):
        for _ in range(2):
            pltpu.make_async_copy(
                src_hbm.at[pl.ds(0, CHUNK)], sidx[q], sem_i[q]
            ).wait()

    def stage(i, jm, first_s=False, idx_next=True, gather_next=True):
        if not first_s:
            drain_s((jm - 1) % NSLOT)
        if gather_next:
            drain_i((jm + NSLOT - 1) % NIB)
            gather((jm + NSLOT - 1) % NIB, (jm + NSLOT - 1) % NSLOT)
        drain_g(jm % NSLOT)
        if idx_next:
            load_idx(i + IDXLA, (jm + IDXLA) % NIB)
        scatter(jm % NIB, jm % NSLOT)

    for i in range(NSLOT - 1):
        pltpu.sync_copy(src_hbm.at[pl.ds(off(i), CHUNK)], sidx[i])
        pltpu.sync_copy(dst_hbm.at[pl.ds(off(i), CHUNK)], didx[i])
        gather(i, i)
    load_idx(NSLOT - 1, NSLOT - 1)
    load_idx(NSLOT, NSLOT)

    z0 = (NSLOT - 1) * CHUNK
    zv = jnp.zeros((LANES,), jnp.float32)

    def zero_body(i, carry):
        for k in range(D // LANES):
            rows[z0 + i, pl.ds(k * LANES, LANES)] = zv
        return carry

    lax.fori_loop(0, 64, zero_body, 0)
    for k in range(9):
        pltpu.sync_copy(
            rows.at[pl.ds(z0, 64)], agg.at[pl.ds(base + k * 64, 64)]
        )
    pltpu.sync_copy(rows.at[pl.ds(z0, 56)], agg.at[pl.ds(base + 576, 56)])
    plsc.subcore_barrier()

    for i in range(2):
        stage(i, i % PERIOD, first_s=(i == 0))

    def loop_body(k, carry):
        for jj in range(PERIOD):
            stage(2 + PERIOD * k + jj, (2 + jj) % PERIOD)
        return carry

    lax.fori_loop(0, 12, loop_body, 0)

    for i in range(74, NCHUNKS):
        stage(
            i,
            i % PERIOD,
            idx_next=(i + IDXLA < NCHUNKS),
            gather_next=(i + NSLOT - 1 < NCHUNKS),
        )
    drain_s((NCHUNKS - 1) % NSLOT)
    plsc.subcore_barrier()

    pltpu.sync_copy(
        agg.at[pl.ds(base, ROWS_PER_TILE)],
        out_hbm.at[c, pl.ds(base, ROWS_PER_TILE)],
    )


def _combine_body(p0_ref, p1_ref, d0_ref, d1_ref, o_ref):
    scale = 1.0 / jnp.maximum(d0_ref[...] + d1_ref[...], 1.0)
    o_ref[...] = (p0_ref[...] + p1_ref[...]) * scale


_CBLK = NA // 8


_combine = pl.pallas_call(
    _combine_body,
    out_shape=jax.ShapeDtypeStruct((NA, D), jnp.float32),
    grid=(8,),
    in_specs=[
        pl.BlockSpec((_CBLK, D), lambda i: (i, 0)),
        pl.BlockSpec((_CBLK, D), lambda i: (i, 0)),
        pl.BlockSpec((_CBLK, 1), lambda i: (i, 0)),
        pl.BlockSpec((_CBLK, 1), lambda i: (i, 0)),
    ],
    out_specs=pl.BlockSpec((_CBLK, D), lambda i: (i, 0)),
)


def kernel(x, edge_index):
    src = edge_index[0].astype(jnp.int32)
    dst = edge_index[1].astype(jnp.int32)
    npad = EP - N_EDGES
    pad_src = N_NODES + (lax.iota(jnp.int32, npad) % PAD_ROWS)
    pad_dst = lax.iota(jnp.int32, npad) % N_NODES
    srcp = jnp.concatenate([src, pad_src])
    dstp = jnp.concatenate([dst, pad_dst])

    h = jnp.concatenate([x, jnp.zeros((PAD_ROWS, D), jnp.float32)], axis=0)

    ones_h = jnp.concatenate(
        [
            jnp.ones((N_NODES, D), jnp.float32),
            jnp.zeros((PAD_ROWS, D), jnp.float32),
        ],
        axis=0,
    )
    dp = _sc_round(ones_h, srcp, dstp)
    d0 = dp[0, :, 0:1]
    d1 = dp[1, :, 0:1]

    for _ in range(NUM_ITERS):
        p = _sc_round(h, srcp, dstp)
        h = _combine(p[0], p[1], d0, d1)

    return h[:N_NODES]
